# baseline (speedup 1.0000x reference)
import functools

import jax
import jax.numpy as jnp
from jax import lax
from jax.experimental import pallas as pl
from jax.experimental.pallas import tpu as pltpu
from jax.experimental.pallas import tpu_sc as plsc

L = 256
D = 32768
NC = 2
NS = 16
NW = NC * NS
RPW = L // NW


NBUF = 3
LOOKAHEAD = 2


def _permute_body(x_hbm, perm_hbm, out_hbm, idx_ref, bufs, gsems, ssems):
    c = lax.axis_index("c")
    s = lax.axis_index("s")
    wid = s * NC + c
    base = wid * RPW

    pltpu.sync_copy(perm_hbm.at[pl.ds(base, RPW)], idx_ref)

    g = [None] * RPW
    st = [None] * RPW
    for k in range(LOOKAHEAD):
        g[k] = pltpu.async_copy(x_hbm.at[idx_ref.at[k]], bufs[k], gsems[k])
    for k in range(RPW):
        sl = k % NBUF
        g[k].wait()
        st[k] = pltpu.async_copy(bufs[sl], out_hbm.at[pl.ds(base + k, 1)],
                                 ssems[sl])
        m = k + LOOKAHEAD
        if m < RPW:
            if m - NBUF >= 0:
                st[m - NBUF].wait()
            g[m] = pltpu.async_copy(x_hbm.at[idx_ref.at[m]],
                                    bufs[m % NBUF], gsems[m % NBUF])
    for k in range(RPW):
        if st[k] is not None and k > RPW - 1 - NBUF:
            st[k].wait()


@functools.partial(
    pl.kernel,
    out_type=jax.ShapeDtypeStruct((L, D), jnp.float32),
    mesh=plsc.VectorSubcoreMesh(core_axis_name="c", subcore_axis_name="s"),
    scratch_types=[
        pltpu.VMEM((RPW, 1), jnp.int32),
        [pltpu.VMEM((1, D), jnp.float32)] * NBUF,
        [pltpu.SemaphoreType.DMA] * NBUF,
        [pltpu.SemaphoreType.DMA] * NBUF,
    ],
)
def _permute(x_hbm, perm_hbm, out_hbm, idx_ref, bufs, gsems, ssems):
    _permute_body(x_hbm, perm_hbm, out_hbm, idx_ref, bufs, gsems, ssems)


def kernel(x, permutations):
    perm2d = permutations.astype(jnp.int32).reshape(L, 1)
    return _permute(x, perm2d)
